# baseline (speedup 1.0000x reference)
import functools

import jax
import jax.numpy as jnp
from jax import lax
from jax.experimental import pallas as pl
from jax.experimental.pallas import tpu as pltpu
from jax.experimental.pallas import tpu_sc as plsc

N = 16384
D = 32
K = 512

NC = 2
NS = 16
L = 16
NW = NC * NS

P = 1
CHUNK = N // P
BLOCK_N = 1024
CH = 128
KH = 256


def _dist_block(x_ref, c_ref, o_ref):
    x = x_ref[...]
    c = c_ref[...]
    s = lax.dot_general(c, x, (((0,), (1,)), ((), ())),
                        preferred_element_type=jnp.float32,
                        precision=lax.Precision.HIGHEST)
    cn = jnp.sum(c * c, axis=0)[:, None]
    o_ref[...] = cn - 2.0 * s


def _sc_argmin(dT_hbm, out_hbm, bufs, outv, sems):
    rows_per_w = CHUNK // NW
    n_piece = (rows_per_w // CH) * (K // KH)
    wid = lax.axis_index("s") * NC + lax.axis_index("c")
    base = wid * rows_per_w

    def start(i):
        chunk, half = divmod(i, K // KH)
        src = dT_hbm.at[pl.ds(half * KH, KH),
                        pl.ds(base + chunk * CH, CH)]
        return pltpu.async_copy(src, bufs.at[i % 2], sems.at[i % 2])

    copies = {0: start(0)}
    carry = None
    for i in range(n_piece):
        chunk, half = divmod(i, K // KH)
        if i + 1 < n_piece:
            copies[i + 1] = start(i + 1)
        copies.pop(i).wait()
        buf = bufs.at[i % 2]
        if half == 0:
            carry = []
            for g in range(CH // L):
                carry.append(jnp.full((L,), jnp.inf, jnp.float32))
                carry.append(jnp.zeros((L,), jnp.float32))
            carry = tuple(carry)

        def body(k, c, _half=half, _buf=buf):
            kf = lax.convert_element_type(k + _half * KH, jnp.float32)
            new = []
            for g in range(CH // L):
                mv, mi = c[2 * g], c[2 * g + 1]
                v = _buf[k, pl.ds(g * L, L)]
                p = v < mv
                new.append(jnp.where(p, v, mv))
                new.append(jnp.where(p, kf, mi))
            return tuple(new)

        carry = lax.fori_loop(0, KH, body, carry)
        if half == K // KH - 1:
            for g in range(CH // L):
                outv[pl.ds(chunk * CH + g * L, L)] = carry[2 * g + 1]
    pltpu.sync_copy(outv, out_hbm.at[pl.ds(base, rows_per_w)])


_SC_MESH = plsc.VectorSubcoreMesh(core_axis_name="c", subcore_axis_name="s",
                                  num_cores=NC, num_subcores=NS)

_sc_argmin_call = functools.partial(
    pl.kernel,
    _sc_argmin,
    out_type=jax.ShapeDtypeStruct((CHUNK,), jnp.float32),
    mesh=_SC_MESH,
    scratch_types=[
        pltpu.VMEM((2, KH, CH), jnp.float32),
        pltpu.VMEM((CHUNK // NW,), jnp.float32),
        pltpu.SemaphoreType.DMA((2,)),
    ],
)()


@jax.jit
def kernel(features, centroids):
    dist_call = pl.pallas_call(
        _dist_block,
        grid=(CHUNK // BLOCK_N,),
        in_specs=[
            pl.BlockSpec((BLOCK_N, D), lambda i: (i, 0)),
            pl.BlockSpec((D, K), lambda i: (0, 0)),
        ],
        out_specs=pl.BlockSpec((K, BLOCK_N), lambda i: (0, i)),
        out_shape=jax.ShapeDtypeStruct((K, CHUNK), jnp.float32),
    )
    outs = []
    for p in range(P):
        xp = lax.slice(features, (p * CHUNK, 0), ((p + 1) * CHUNK, D))
        dTp = dist_call(xp, centroids)
        outs.append(_sc_argmin_call(dTp))
    return jnp.concatenate(outs)[:, None]


# speedup vs baseline: 1.0184x; 1.0184x over previous
import functools

import jax
import jax.numpy as jnp
from jax import lax
from jax.experimental import pallas as pl
from jax.experimental.pallas import tpu as pltpu
from jax.experimental.pallas import tpu_sc as plsc

N = 16384
D = 32
K = 512

NC = 2
NS = 16
L = 16
NW = NC * NS

P = 1
CHUNK = N // P
BLOCK_N = 2048
CH = 128
KH = 256


def _dist_block(x_ref, c_ref, o_ref):
    x = x_ref[...]
    c = c_ref[...]
    s = lax.dot_general(c, x, (((0,), (1,)), ((), ())),
                        preferred_element_type=jnp.float32,
                        precision=lax.Precision.HIGHEST)
    cn = jnp.sum(c * c, axis=0)[:, None]
    o_ref[...] = cn - 2.0 * s


def _sc_argmin(dT_hbm, out_hbm, bufs, outv, sems):
    rows_per_w = CHUNK // NW
    n_piece = (rows_per_w // CH) * (K // KH)
    wid = lax.axis_index("s") * NC + lax.axis_index("c")
    base = wid * rows_per_w

    def start(i):
        chunk, half = divmod(i, K // KH)
        src = dT_hbm.at[pl.ds(half * KH, KH),
                        pl.ds(base + chunk * CH, CH)]
        return pltpu.async_copy(src, bufs.at[i % 2], sems.at[i % 2])

    copies = {0: start(0)}
    carry = None
    for i in range(n_piece):
        chunk, half = divmod(i, K // KH)
        if i + 1 < n_piece:
            copies[i + 1] = start(i + 1)
        copies.pop(i).wait()
        buf = bufs.at[i % 2]
        if half == 0:
            carry = []
            for g in range(CH // L):
                carry.append(jnp.full((L,), jnp.inf, jnp.float32))
                carry.append(jnp.zeros((L,), jnp.float32))
            carry = tuple(carry)

        def body(k, c, _half=half, _buf=buf):
            kf = lax.convert_element_type(k + _half * KH, jnp.float32)
            new = []
            for g in range(CH // L):
                mv, mi = c[2 * g], c[2 * g + 1]
                v = _buf[k, pl.ds(g * L, L)]
                p = v < mv
                new.append(jnp.where(p, v, mv))
                new.append(jnp.where(p, kf, mi))
            return tuple(new)

        carry = lax.fori_loop(0, KH, body, carry)
        if half == K // KH - 1:
            for g in range(CH // L):
                outv[pl.ds(chunk * CH + g * L, L)] = carry[2 * g + 1]
    pltpu.sync_copy(outv, out_hbm.at[pl.ds(base, rows_per_w)])


_SC_MESH = plsc.VectorSubcoreMesh(core_axis_name="c", subcore_axis_name="s",
                                  num_cores=NC, num_subcores=NS)

_sc_argmin_call = functools.partial(
    pl.kernel,
    _sc_argmin,
    out_type=jax.ShapeDtypeStruct((CHUNK,), jnp.float32),
    mesh=_SC_MESH,
    scratch_types=[
        pltpu.VMEM((2, KH, CH), jnp.float32),
        pltpu.VMEM((CHUNK // NW,), jnp.float32),
        pltpu.SemaphoreType.DMA((2,)),
    ],
)()


@jax.jit
def kernel(features, centroids):
    dist_call = pl.pallas_call(
        _dist_block,
        grid=(CHUNK // BLOCK_N,),
        in_specs=[
            pl.BlockSpec((BLOCK_N, D), lambda i: (i, 0)),
            pl.BlockSpec((D, K), lambda i: (0, 0)),
        ],
        out_specs=pl.BlockSpec((K, BLOCK_N), lambda i: (0, i)),
        out_shape=jax.ShapeDtypeStruct((K, CHUNK), jnp.float32),
    )
    outs = []
    for p in range(P):
        xp = lax.slice(features, (p * CHUNK, 0), ((p + 1) * CHUNK, D))
        dTp = dist_call(xp, centroids)
        outs.append(_sc_argmin_call(dTp))
    return jnp.concatenate(outs)[:, None]
